# baseline (speedup 1.0000x reference)
import functools

import jax
import jax.numpy as jnp
from jax import lax
from jax.experimental import pallas as pl
from jax.experimental.pallas import tpu as pltpu
from jax.experimental.pallas import tpu_sc as plsc

B = 16384
D = 64
H = 128
CH = 128
NR = B // CH

try:
    _info = plsc.get_sparse_core_info()
    _NC, _NS = _info.num_cores, _info.num_subcores
except Exception:
    _NC, _NS = 2, 16
_NW = _NC * _NS
_NCH = NR // _NW


CHUNK = B // _NW


def _gather_body(uidx_hbm, bidx_hbm, U_hbm, M_hbm, uout, mout,
                 uidx_v, bidx_v, sem):
    wid = lax.axis_index("s") * _NC + lax.axis_index("c")
    base = wid * CHUNK
    pltpu.sync_copy(uidx_hbm.at[pl.ds(base, CHUNK)], uidx_v)
    pltpu.sync_copy(bidx_hbm.at[pl.ds(base, CHUNK)], bidx_v)

    def fire(g, carry):
        uv = uidx_v[pl.ds(g * 16, 16)]
        bv = bidx_v[pl.ds(g * 16, 16)]
        for k in range(16):
            j = g * 16 + k
            pltpu.async_copy(U_hbm.at[pl.ds(uv[k], 1)],
                             uout.at[pl.ds(base + j, 1)], sem)
            pltpu.async_copy(M_hbm.at[pl.ds(bv[k], 1)],
                             mout.at[pl.ds(base + j, 1)], sem)
        return carry

    lax.fori_loop(0, CHUNK // 16, fire, 0)

    def drain(j, carry):
        pltpu.make_async_copy(
            U_hbm.at[pl.ds(0, 1)], uout.at[pl.ds(base + j, 1)], sem).wait()
        pltpu.make_async_copy(
            M_hbm.at[pl.ds(0, 1)], mout.at[pl.ds(base + j, 1)], sem).wait()
        return carry

    lax.fori_loop(0, CHUNK, drain, 0)


@functools.cache
def _make_gather():
    return pl.kernel(
        _gather_body,
        mesh=plsc.VectorSubcoreMesh(core_axis_name="c", subcore_axis_name="s"),
        out_type=[
            jax.ShapeDtypeStruct((B, D), jnp.float32),
            jax.ShapeDtypeStruct((B, D), jnp.float32),
        ],
        scratch_types=[
            pltpu.VMEM((CHUNK,), jnp.int32),
            pltpu.VMEM((CHUNK,), jnp.int32),
            pltpu.SemaphoreType.DMA,
        ],
    )


BLK = 1024


def _mlp_body(u_ref, m_ref, w1a_ref, w1b_ref, b1_ref, w2_ref, b2_ref, o_ref):
    x = jnp.dot(u_ref[...], w1a_ref[...], preferred_element_type=jnp.float32)
    x = x + jnp.dot(m_ref[...], w1b_ref[...], preferred_element_type=jnp.float32)
    x = jnp.maximum(x + b1_ref[...], 0.0)
    y = jnp.sum(x * w2_ref[...], axis=1, keepdims=True) + b2_ref[0, 0]
    o_ref[...] = 1.0 / (1.0 + jnp.exp(-y))


def _mlp(u_emb, m_emb, w1a, w1b, b1, w2, b2):
    return pl.pallas_call(
        _mlp_body,
        grid=(B // BLK,),
        in_specs=[
            pl.BlockSpec((BLK, D), lambda i: (i, 0)),
            pl.BlockSpec((BLK, D), lambda i: (i, 0)),
            pl.BlockSpec((D, H), lambda i: (0, 0)),
            pl.BlockSpec((D, H), lambda i: (0, 0)),
            pl.BlockSpec((1, H), lambda i: (0, 0)),
            pl.BlockSpec((1, H), lambda i: (0, 0)),
            pl.BlockSpec(memory_space=pltpu.SMEM),
        ],
        out_specs=pl.BlockSpec((BLK, 1), lambda i: (i, 0)),
        out_shape=jax.ShapeDtypeStruct((B, 1), jnp.float32),
    )(u_emb, m_emb, w1a, w1b, b1, w2, b2)


def kernel(users, books, U, M, W1, b1, W2, b2):
    u_emb, m_emb = _make_gather()(users.astype(jnp.int32),
                                  books.astype(jnp.int32), U, M)
    w1a = W1[:, :D].T
    w1b = W1[:, D:].T
    return _mlp(u_emb, m_emb, w1a, w1b,
                b1.reshape(1, H), W2, b2.reshape(1, 1))


# speedup vs baseline: 1.6676x; 1.6676x over previous
import functools

import jax
import jax.numpy as jnp
from jax import lax
from jax.experimental import pallas as pl
from jax.experimental.pallas import tpu as pltpu
from jax.experimental.pallas import tpu_sc as plsc

B = 16384
D = 64
H = 128
CH = 128
NR = B // CH

try:
    _info = plsc.get_sparse_core_info()
    _NC, _NS = _info.num_cores, _info.num_subcores
except Exception:
    _NC, _NS = 2, 16
_NW = _NC * _NS
_NCH = NR // _NW


CHUNK = B // _NW


def _gather_one(idx_v, tbl_hbm, out_hbm, rows_v, sem, base):
    def fire(g, carry):
        iv = idx_v[pl.ds(g * 16, 16)]
        for k in range(16):
            j = g * 16 + k
            pltpu.async_copy(tbl_hbm.at[pl.ds(iv[k], 1)],
                             rows_v.at[pl.ds(j, 1)], sem)
        return carry

    lax.fori_loop(0, CHUNK // 16, fire, 0)

    def drain(j, carry):
        pltpu.make_async_copy(
            tbl_hbm.at[pl.ds(0, 1)], rows_v.at[pl.ds(j, 1)], sem).wait()
        return carry

    lax.fori_loop(0, CHUNK, drain, 0)
    pltpu.sync_copy(rows_v, out_hbm.at[pl.ds(base, CHUNK)])


def _gather_body(uidx_hbm, bidx_hbm, U_hbm, M_hbm, uout, mout,
                 uidx_v, bidx_v, rows_v, sem):
    wid = lax.axis_index("s") * _NC + lax.axis_index("c")
    base = wid * CHUNK
    pltpu.sync_copy(uidx_hbm.at[pl.ds(base, CHUNK)], uidx_v)
    pltpu.sync_copy(bidx_hbm.at[pl.ds(base, CHUNK)], bidx_v)
    _gather_one(uidx_v, U_hbm, uout, rows_v, sem, base)
    _gather_one(bidx_v, M_hbm, mout, rows_v, sem, base)


@functools.cache
def _make_gather():
    return pl.kernel(
        _gather_body,
        mesh=plsc.VectorSubcoreMesh(core_axis_name="c", subcore_axis_name="s"),
        out_type=[
            jax.ShapeDtypeStruct((B, D), jnp.float32),
            jax.ShapeDtypeStruct((B, D), jnp.float32),
        ],
        scratch_types=[
            pltpu.VMEM((CHUNK,), jnp.int32),
            pltpu.VMEM((CHUNK,), jnp.int32),
            pltpu.VMEM((CHUNK, D), jnp.float32),
            pltpu.SemaphoreType.DMA,
        ],
    )


BLK = 1024


def _mlp_body(u_ref, m_ref, w1a_ref, w1b_ref, b1_ref, w2_ref, b2_ref, o_ref):
    x = jnp.dot(u_ref[...], w1a_ref[...], preferred_element_type=jnp.float32)
    x = x + jnp.dot(m_ref[...], w1b_ref[...], preferred_element_type=jnp.float32)
    x = jnp.maximum(x + b1_ref[...], 0.0)
    y = jnp.sum(x * w2_ref[...], axis=1, keepdims=True) + b2_ref[0, 0]
    o_ref[...] = 1.0 / (1.0 + jnp.exp(-y))


def _mlp(u_emb, m_emb, w1a, w1b, b1, w2, b2):
    return pl.pallas_call(
        _mlp_body,
        grid=(B // BLK,),
        in_specs=[
            pl.BlockSpec((BLK, D), lambda i: (i, 0)),
            pl.BlockSpec((BLK, D), lambda i: (i, 0)),
            pl.BlockSpec((D, H), lambda i: (0, 0)),
            pl.BlockSpec((D, H), lambda i: (0, 0)),
            pl.BlockSpec((1, H), lambda i: (0, 0)),
            pl.BlockSpec((1, H), lambda i: (0, 0)),
            pl.BlockSpec(memory_space=pltpu.SMEM),
        ],
        out_specs=pl.BlockSpec((BLK, 1), lambda i: (i, 0)),
        out_shape=jax.ShapeDtypeStruct((B, 1), jnp.float32),
    )(u_emb, m_emb, w1a, w1b, b1, w2, b2)


def kernel(users, books, U, M, W1, b1, W2, b2):
    u_emb, m_emb = _make_gather()(users.astype(jnp.int32),
                                  books.astype(jnp.int32), U, M)
    w1a = W1[:, :D].T
    w1b = W1[:, D:].T
    return _mlp(u_emb, m_emb, w1a, w1b,
                b1.reshape(1, H), W2, b2.reshape(1, 1))
